# Initial kernel scaffold; baseline (speedup 1.0000x reference)
import jax
import jax.numpy as jnp
from jax.experimental import pallas as pl


def kernel(timestep, prompt_embeddings, W1, b1, W2, b2, Wg, bg, w_noise):
    raise NotImplementedError("write your pallas kernel here")


# speedup vs baseline: 1.5322x; 1.5322x over previous
import functools

import jax
import jax.numpy as jnp
from jax import lax
from jax.experimental import pallas as pl
from jax.experimental.pallas import tpu as pltpu

B = 32
H = 1024
T = 2048
DEPTH = 28
K = 512

TBLK = 256
SBLK = 128
HBLK = 512


def _embed_body(t_ref, w1_ref, b1_ref, w2_ref, b2_ref, out_ref):
    h = t_ref[...] * w1_ref[...] + b1_ref[...]
    h = h * jax.nn.sigmoid(h)
    out_ref[...] = (
        jnp.dot(
            h.astype(jnp.bfloat16),
            w2_ref[...].astype(jnp.bfloat16),
            preferred_element_type=jnp.float32,
        )
        + b2_ref[...]
    )


def _t_embed(t, W1, b1r, W2, b2r):
    return pl.pallas_call(
        _embed_body,
        out_shape=jax.ShapeDtypeStruct((B, H), jnp.float32),
    )(t, W1, b1r, W2, b2r)


def _noisy_body(te_ref, wg_ref, wgl_ref, bg_ref, wn_ref, nz_ref, step_ref, out_ref):
    te = te_ref[...].astype(jnp.bfloat16)
    step_b = step_ref[...].astype(jnp.bfloat16).astype(jnp.float32)
    wgl_b = wgl_ref[...].astype(jnp.bfloat16).astype(jnp.float32)
    clean = (
        jnp.dot(te, wg_ref[...].astype(jnp.bfloat16), preferred_element_type=jnp.float32)
        + step_b * wgl_b
        + bg_ref[...]
    )
    raw = jnp.dot(te, wn_ref[...].astype(jnp.bfloat16), preferred_element_type=jnp.float32)
    std = jnp.maximum(raw, 0.0) + jnp.log1p(jnp.exp(-jnp.abs(raw))) + 0.01
    out_ref[...] = clean + nz_ref[...] * std


def _noisy_logits(t_embed, Wg_main, wg_last, bgr, w_noise, noise, step_f):
    grid = (T // TBLK,)
    return pl.pallas_call(
        _noisy_body,
        grid=grid,
        in_specs=[
            pl.BlockSpec((B, H), lambda i: (0, 0)),
            pl.BlockSpec((H, TBLK), lambda i: (0, i)),
            pl.BlockSpec((1, TBLK), lambda i: (0, i)),
            pl.BlockSpec((1, TBLK), lambda i: (0, i)),
            pl.BlockSpec((H, TBLK), lambda i: (0, i)),
            pl.BlockSpec((B, TBLK), lambda i: (0, i)),
            pl.BlockSpec((1, 1), lambda i: (0, 0)),
        ],
        out_specs=pl.BlockSpec((B, TBLK), lambda i: (0, i)),
        out_shape=jax.ShapeDtypeStruct((B, T), jnp.float32),
    )(t_embed, Wg_main, wg_last, bgr, w_noise, noise, step_f)


def _gates_body(x_ref, out_ref):
    x = x_ref[...]
    s = lax.bitcast_convert_type(x, jnp.int32)
    m = lax.shift_right_arithmetic(s, 31)
    keys = lax.bitcast_convert_type(
        s ^ (m & jnp.int32(0x7FFFFFFF)) ^ jnp.int32(-0x80000000), jnp.uint32
    )

    def bit_step(i, t):
        cand = t | (jnp.uint32(1) << (jnp.uint32(31) - i.astype(jnp.uint32)))
        cnt = jnp.sum((keys >= cand).astype(jnp.int32), axis=1, keepdims=True)
        return jnp.where(cnt >= K, cand, t)

    thr = lax.fori_loop(0, 32, bit_step, jnp.zeros((B, 1), jnp.uint32))

    gt = keys > thr
    eq = keys == thr
    n_gt = jnp.sum(gt.astype(jnp.int32), axis=1, keepdims=True)
    tie_rank = eq.astype(jnp.int32)
    d = 1
    while d < T:
        shifted = jnp.concatenate(
            [jnp.zeros((B, d), jnp.int32), tie_rank[:, : T - d]], axis=1
        )
        tie_rank = tie_rank + shifted
        d *= 2
    sel = gt | (eq & (tie_rank <= (K - n_gt)))

    mx = jnp.max(x, axis=1, keepdims=True)
    e = jnp.where(sel, jnp.exp(x - mx), 0.0)
    out_ref[...] = e / jnp.sum(e, axis=1, keepdims=True)


def _gates(noisy):
    return pl.pallas_call(
        _gates_body,
        out_shape=jax.ShapeDtypeStruct((B, T), jnp.float32),
    )(noisy)


def _scale_body(step_ref, p_ref, g_ref, out_ref):
    del step_ref
    out_ref[...] = g_ref[...][:, :, None] * p_ref[...]


def _scale(timestep, prompt_embeddings, gates):
    grid = (T // SBLK, H // HBLK)
    return pl.pallas_call(
        _scale_body,
        grid_spec=pltpu.PrefetchScalarGridSpec(
            num_scalar_prefetch=1,
            grid=grid,
            in_specs=[
                pl.BlockSpec((1, SBLK, HBLK), lambda i, j, step: (step[0], i, j)),
                pl.BlockSpec((B, SBLK), lambda i, j, step: (0, i)),
            ],
            out_specs=pl.BlockSpec((B, SBLK, HBLK), lambda i, j, step: (0, i, j)),
        ),
        out_shape=jax.ShapeDtypeStruct((B, T, H), jnp.float32),
    )(timestep, prompt_embeddings, gates)


def kernel(timestep, prompt_embeddings, W1, b1, W2, b2, Wg, bg, w_noise):
    t = timestep.astype(jnp.float32).reshape(B, 1)
    step_f = timestep[0].astype(jnp.float32).reshape(1, 1)
    noise = jax.random.normal(jax.random.key(1234), (B, T), jnp.float32)

    t_embed = _t_embed(t, W1, b1.reshape(1, H), W2, b2.reshape(1, H))
    noisy = _noisy_logits(
        t_embed, Wg[:H], Wg[H:].reshape(1, T), bg.reshape(1, T), w_noise, noise, step_f
    )
    gates = _gates(noisy)
    prompts = _scale(timestep, prompt_embeddings, gates)
    return prompts, t_embed
